# baseline (speedup 1.0000x reference)
import functools

import jax
import jax.numpy as jnp
from jax import lax
from jax.experimental import pallas as pl
from jax.experimental.pallas import tpu as pltpu
from jax.experimental.pallas import tpu_sc as plsc

N = 10000
E = 320000
D = 128
DH = 64

NUM_CORES = 2
NUM_SUBCORES = 16
CHUNK = 128
NUM_CHUNKS = E // CHUNK
MAXC = NUM_CHUNKS // NUM_SUBCORES + 1
ROWS_PER_SUBCORE = N // NUM_SUBCORES
NBUF = 4
AHEAD = NBUF - 1


def _sc_aggregate(xs, src_both, dst2):
    @functools.partial(
        pl.kernel,
        out_type=(
            jax.ShapeDtypeStruct((NUM_CORES, N, DH), jnp.float32),
            jax.ShapeDtypeStruct((NUM_SUBCORES, N), jnp.float32),
        ),
        mesh=plsc.VectorSubcoreMesh(core_axis_name="c", subcore_axis_name="s"),
        scratch_types=[
            pltpu.VMEM((MAXC, CHUNK), jnp.int32),
            pltpu.VMEM((MAXC, CHUNK), jnp.int32),
            pltpu.VMEM((NBUF, CHUNK, DH), jnp.float32),
            pltpu.VMEM((N,), jnp.float32),
            pltpu.VMEM_SHARED((N, DH), jnp.float32),
            pltpu.SemaphoreType.DMA((NBUF,)),
            pltpu.SemaphoreType.DMA((NBUF,)),
        ],
        compiler_params=pltpu.CompilerParams(use_tc_tiling_on_sc=False,
                                             needs_layout_passes=False),
    )
    def body(xs_hbm, src_hbm, dst_hbm, out_hbm, hist_hbm,
             src_v, dst_v, rows, hist, accum, sem_g, sem_s):
        cid = lax.axis_index("c")
        sid = lax.axis_index("s")

        lo = sid * NUM_CHUNKS // NUM_SUBCORES
        hi = (sid + 1) * NUM_CHUNKS // NUM_SUBCORES
        n = hi - lo

        pltpu.sync_copy(src_hbm.at[cid, pl.ds(lo, MAXC)], src_v)
        pltpu.sync_copy(dst_hbm.at[pl.ds(lo, MAXC)], dst_v)

        def gather_start(c, b):
            pltpu.async_copy(xs_hbm.at[src_v.at[c]], rows.at[b], sem_g.at[b])

        def gather_wait(c, b):
            pltpu.make_async_copy(
                xs_hbm.at[src_v.at[c]], rows.at[b], sem_g.at[b]).wait()

        def scatter_start(c, b):
            pltpu.async_copy(rows.at[b], accum.at[dst_v.at[c]], sem_s.at[b],
                             add=True)

        def scatter_wait(c, b):
            pltpu.make_async_copy(
                rows.at[b], accum.at[dst_v.at[c]], sem_s.at[b]).wait()

        zvec = jnp.zeros((16,), jnp.float32)
        ones16 = jnp.ones((16,), jnp.float32)

        def hzero(q, carry):
            hist[pl.ds(q * 16, 16)] = zvec
            return carry

        lax.fori_loop(0, N // 16, hzero, 0)

        def zstore(q, carry):
            r = q // (DH // 16)
            col = q % (DH // 16)
            rows[0, r, pl.ds(col * 16, 16)] = zvec
            return carry

        lax.fori_loop(0, CHUNK * (DH // 16), zstore, 0)
        row0 = sid * ROWS_PER_SUBCORE
        for z in range(4):
            pltpu.sync_copy(rows.at[0].at[pl.ds(0, CHUNK)],
                            accum.at[pl.ds(row0 + z * CHUNK, CHUNK)])
        pltpu.sync_copy(rows.at[0].at[pl.ds(0, ROWS_PER_SUBCORE - 4 * CHUNK)],
                        accum.at[pl.ds(row0 + 4 * CHUNK,
                                       ROWS_PER_SUBCORE - 4 * CHUNK)])

        for k in range(AHEAD):
            gather_start(k, k)
        plsc.subcore_barrier()

        def step(jj, carry):
            for b in range(NBUF):
                c = jj * NBUF + b

                @pl.when(c < n)
                def _():
                    cg = c + AHEAD
                    bg = (b + AHEAD) % NBUF

                    @pl.when(cg < n)
                    def _():
                        @pl.when(cg - NBUF >= 0)
                        def _():
                            scatter_wait(cg - NBUF, bg)
                        gather_start(cg, bg)

                    gather_wait(c, b)
                    scatter_start(c, b)

                    for k in range(CHUNK // 16):
                        idx = dst_v[c, pl.ds(k * 16, 16)]
                        plsc.addupdate_scatter(hist, [idx], ones16)

            return carry

        lax.fori_loop(0, (n + NBUF - 1) // NBUF, step, 0)

        for b in range(NBUF):
            scatter_wait(0, b)

        plsc.subcore_barrier()

        pltpu.sync_copy(
            accum.at[pl.ds(row0, ROWS_PER_SUBCORE)],
            out_hbm.at[cid, pl.ds(row0, ROWS_PER_SUBCORE)],
        )

        @pl.when(cid == 0)
        def _():
            pltpu.sync_copy(hist, hist_hbm.at[sid])

    return body(xs, src_both, dst2)


def _tc_self_body(x_ref, ws_ref, b_ref, out_ref):
    out_ref[...] = lax.dot_general(
        x_ref[...], ws_ref[...], (((1,), (1,)), ((), ())),
        preferred_element_type=jnp.float32) + b_ref[...]


def _tc_self(x, w_self, b_self):
    R = 5000
    return pl.pallas_call(
        _tc_self_body,
        grid=(N // R,),
        in_specs=[
            pl.BlockSpec((R, D), lambda i: (i, 0)),
            pl.BlockSpec((D, D), lambda i: (0, 0)),
            pl.BlockSpec((1, D), lambda i: (0, 0)),
        ],
        out_specs=pl.BlockSpec((R, D), lambda i: (i, 0)),
        out_shape=jax.ShapeDtypeStruct((N, D), jnp.float32),
    )(x, w_self, b_self)


def _tc_finish_body(partial_ref, hist_ref, hs_ref, wn_ref, out_ref):
    sums = jnp.concatenate([partial_ref[0], partial_ref[1]], axis=1)
    deg = jnp.sum(hist_ref[...], axis=1, keepdims=True)
    h = sums / jnp.maximum(deg, 1.0)
    hn = lax.dot_general(h, wn_ref[...], (((1,), (1,)), ((), ())),
                         preferred_element_type=jnp.float32)
    out_ref[...] = hn + hs_ref[...]


def _tc_finish(partial, hist, hs, w_neigh):
    R = 5000
    grid = (N // R,)
    return pl.pallas_call(
        _tc_finish_body,
        grid=grid,
        in_specs=[
            pl.BlockSpec((NUM_CORES, R, DH), lambda i: (0, i, 0)),
            pl.BlockSpec((R, NUM_SUBCORES), lambda i: (i, 0)),
            pl.BlockSpec((R, D), lambda i: (i, 0)),
            pl.BlockSpec((D, D), lambda i: (0, 0)),
        ],
        out_specs=pl.BlockSpec((R, D), lambda i: (i, 0)),
        out_shape=jax.ShapeDtypeStruct((N, D), jnp.float32),
    )(partial, hist, hs, w_neigh)


def kernel(x, edge_index, W_neigh, W_self, b_self):
    src2 = edge_index[0].reshape(NUM_CHUNKS, CHUNK)
    dst2 = edge_index[1].reshape(NUM_CHUNKS, CHUNK)
    src_both = jnp.stack([src2 * 2, src2 * 2 + 1])
    xs = x.reshape(NUM_CORES * N, DH)
    partial, hist = _sc_aggregate(xs, src_both, dst2)
    hs = _tc_self(x, W_self, b_self.reshape(1, D))
    return _tc_finish(partial, hist.T, hs, W_neigh)


# speedup vs baseline: 1.0158x; 1.0158x over previous
import functools

import jax
import jax.numpy as jnp
from jax import lax
from jax.experimental import pallas as pl
from jax.experimental.pallas import tpu as pltpu
from jax.experimental.pallas import tpu_sc as plsc

N = 10000
E = 320000
D = 128
DH = 64

NUM_CORES = 2
NUM_SUBCORES = 16
CHUNK = 128
NUM_CHUNKS = E // CHUNK
MAXC = NUM_CHUNKS // NUM_SUBCORES + 1
ROWS_PER_SUBCORE = N // NUM_SUBCORES
NBUF = 4
AHEAD = NBUF - 1


def _sc_aggregate(xs, src_both, dst2):
    @functools.partial(
        pl.kernel,
        out_type=(
            jax.ShapeDtypeStruct((NUM_CORES, N, DH), jnp.float32),
            jax.ShapeDtypeStruct((NUM_SUBCORES, N), jnp.float32),
        ),
        mesh=plsc.VectorSubcoreMesh(core_axis_name="c", subcore_axis_name="s"),
        scratch_types=[
            pltpu.VMEM((MAXC, CHUNK), jnp.int32),
            pltpu.VMEM((MAXC, CHUNK), jnp.int32),
            pltpu.VMEM((NBUF, CHUNK, DH), jnp.float32),
            pltpu.VMEM((N,), jnp.float32),
            pltpu.VMEM_SHARED((N, DH), jnp.float32),
            pltpu.SemaphoreType.DMA((NBUF,)),
            pltpu.SemaphoreType.DMA((NBUF,)),
        ],
        compiler_params=pltpu.CompilerParams(use_tc_tiling_on_sc=False,
                                             needs_layout_passes=False),
    )
    def body(xs_hbm, src_hbm, dst_hbm, out_hbm, hist_hbm,
             src_v, dst_v, rows, hist, accum, sem_g, sem_s):
        cid = lax.axis_index("c")
        sid = lax.axis_index("s")

        lo = sid * NUM_CHUNKS // NUM_SUBCORES
        hi = (sid + 1) * NUM_CHUNKS // NUM_SUBCORES
        n = hi - lo

        pltpu.sync_copy(src_hbm.at[cid, pl.ds(lo, MAXC)], src_v)
        pltpu.sync_copy(dst_hbm.at[pl.ds(lo, MAXC)], dst_v)

        def gather_start(c, b):
            pltpu.async_copy(xs_hbm.at[src_v.at[c]], rows.at[b], sem_g.at[b])

        def gather_wait(c, b):
            pltpu.make_async_copy(
                xs_hbm.at[src_v.at[c]], rows.at[b], sem_g.at[b]).wait()

        def scatter_start(c, b):
            pltpu.async_copy(rows.at[b], accum.at[dst_v.at[c]], sem_s.at[b],
                             add=True)

        def scatter_wait(c, b):
            pltpu.make_async_copy(
                rows.at[b], accum.at[dst_v.at[c]], sem_s.at[b]).wait()

        zvec = jnp.zeros((16,), jnp.float32)
        ones16 = jnp.ones((16,), jnp.float32)

        for k in range(AHEAD):
            gather_start(k, k)

        def hzero(q, carry):
            hist[pl.ds(q * 16, 16)] = zvec
            return carry

        lax.fori_loop(0, N // 16, hzero, 0)

        def zstore(q, carry):
            r = q // (DH // 16)
            col = q % (DH // 16)
            rows[AHEAD, r, pl.ds(col * 16, 16)] = zvec
            return carry

        lax.fori_loop(0, CHUNK * (DH // 16), zstore, 0)
        row0 = sid * ROWS_PER_SUBCORE
        for z in range(4):
            pltpu.sync_copy(rows.at[AHEAD].at[pl.ds(0, CHUNK)],
                            accum.at[pl.ds(row0 + z * CHUNK, CHUNK)])
        pltpu.sync_copy(
            rows.at[AHEAD].at[pl.ds(0, ROWS_PER_SUBCORE - 4 * CHUNK)],
            accum.at[pl.ds(row0 + 4 * CHUNK, ROWS_PER_SUBCORE - 4 * CHUNK)])

        plsc.subcore_barrier()

        def step(jj, carry):
            for b in range(NBUF):
                c = jj * NBUF + b

                @pl.when(c < n)
                def _():
                    cg = c + AHEAD
                    bg = (b + AHEAD) % NBUF

                    @pl.when(cg < n)
                    def _():
                        @pl.when(cg - NBUF >= 0)
                        def _():
                            scatter_wait(cg - NBUF, bg)
                        gather_start(cg, bg)

                    gather_wait(c, b)
                    scatter_start(c, b)

                    for k in range(CHUNK // 16):
                        idx = dst_v[c, pl.ds(k * 16, 16)]
                        plsc.addupdate_scatter(hist, [idx], ones16)

            return carry

        lax.fori_loop(0, (n + NBUF - 1) // NBUF, step, 0)

        for b in range(NBUF):
            scatter_wait(0, b)

        plsc.subcore_barrier()

        pltpu.sync_copy(
            accum.at[pl.ds(row0, ROWS_PER_SUBCORE)],
            out_hbm.at[cid, pl.ds(row0, ROWS_PER_SUBCORE)],
        )

        @pl.when(cid == 0)
        def _():
            pltpu.sync_copy(hist, hist_hbm.at[sid])

    return body(xs, src_both, dst2)


def _tc_self_body(x_ref, ws_ref, b_ref, out_ref):
    out_ref[...] = lax.dot_general(
        x_ref[...], ws_ref[...], (((1,), (1,)), ((), ())),
        preferred_element_type=jnp.float32) + b_ref[...]


def _tc_self(x, w_self, b_self):
    R = 2000
    return pl.pallas_call(
        _tc_self_body,
        grid=(N // R,),
        in_specs=[
            pl.BlockSpec((R, D), lambda i: (i, 0)),
            pl.BlockSpec((D, D), lambda i: (0, 0)),
            pl.BlockSpec((1, D), lambda i: (0, 0)),
        ],
        out_specs=pl.BlockSpec((R, D), lambda i: (i, 0)),
        out_shape=jax.ShapeDtypeStruct((N, D), jnp.float32),
    )(x, w_self, b_self)


def _tc_finish_body(partial_ref, hist_ref, hs_ref, wn_ref, out_ref):
    sums = jnp.concatenate([partial_ref[0], partial_ref[1]], axis=1)
    deg = jnp.sum(hist_ref[...], axis=1, keepdims=True)
    h = sums / jnp.maximum(deg, 1.0)
    hn = lax.dot_general(h, wn_ref[...], (((1,), (1,)), ((), ())),
                         preferred_element_type=jnp.float32)
    out_ref[...] = hn + hs_ref[...]


def _tc_finish(partial, hist, hs, w_neigh):
    R = 2000
    grid = (N // R,)
    return pl.pallas_call(
        _tc_finish_body,
        grid=grid,
        in_specs=[
            pl.BlockSpec((NUM_CORES, R, DH), lambda i: (0, i, 0)),
            pl.BlockSpec((R, NUM_SUBCORES), lambda i: (i, 0)),
            pl.BlockSpec((R, D), lambda i: (i, 0)),
            pl.BlockSpec((D, D), lambda i: (0, 0)),
        ],
        out_specs=pl.BlockSpec((R, D), lambda i: (i, 0)),
        out_shape=jax.ShapeDtypeStruct((N, D), jnp.float32),
    )(partial, hist, hs, w_neigh)


def kernel(x, edge_index, W_neigh, W_self, b_self):
    src2 = edge_index[0].reshape(NUM_CHUNKS, CHUNK)
    dst2 = edge_index[1].reshape(NUM_CHUNKS, CHUNK)
    src_both = jnp.stack([src2 * 2, src2 * 2 + 1])
    xs = x.reshape(NUM_CORES * N, DH)
    partial, hist = _sc_aggregate(xs, src_both, dst2)
    hs = _tc_self(x, W_self, b_self.reshape(1, D))
    return _tc_finish(partial, hist.T, hs, W_neigh)
